# Initial kernel scaffold; baseline (speedup 1.0000x reference)
import jax
import jax.numpy as jnp
from jax.experimental import pallas as pl


def kernel(x, centers, transform_inds):
    raise NotImplementedError("write your pallas kernel here")


# speedup vs baseline: 1.3347x; 1.3347x over previous
import functools

import jax
import jax.numpy as jnp
from jax import lax
from jax.experimental import pallas as pl
from jax.experimental.pallas import tpu as pltpu
from jax.experimental.pallas import tpu_sc as plsc

_B = 16384
_C = 1000
_CP = 1024
_NC = 2
_NS = 16
_NW = _NC * _NS
_RPW = _B // _NW
_L = 16
_G = 4
_MARGIN = 1.0
_INF = float("inf")


def _sc_body(x_hbm, c_hbm, t_hbm, out_hbm, x_v, t_v, c_v, o_v):
    wid = lax.axis_index("s") * _NC + lax.axis_index("c")
    base = wid * _RPW
    pltpu.sync_copy(x_hbm.at[pl.ds(base, _RPW)], x_v)
    pltpu.sync_copy(t_hbm.at[pl.ds(base, _RPW)], t_v)
    pltpu.sync_copy(c_hbm, c_v)

    acc = jnp.zeros((_L,), jnp.float32)
    for g in range(_RPW // (_L * _G)):
        xs = [x_v[pl.ds((g * _G + k) * _L, _L)] for k in range(_G)]
        ts = [t_v[pl.ds((g * _G + k) * _L, _L)] for k in range(_G)]

        def jbody(jj, accs, xs=xs, ts=ts):
            accs = list(accs)
            cblk = c_v[pl.ds(jj * _L, _L)]
            jbase = jj * _L
            for u in range(_L):
                cj = cblk[u]
                j = jbase + u
                for k in range(_G):
                    d = jnp.abs(xs[k] - cj)
                    d = jnp.where(ts[k] == j, _INF, d)
                    accs[k] = jnp.minimum(accs[k], d)
            return tuple(accs)

        init = tuple(jnp.full((_L,), _INF, jnp.float32) for _ in range(_G))
        pushes = lax.fori_loop(0, _CP // _L, jbody, init)
        for k in range(_G):
            own = plsc.load_gather(c_v, [ts[k]])
            pull = jnp.abs(xs[k] - own) + _MARGIN
            acc = acc + jnp.maximum(pull - pushes[k], 0.0)

    o_v[...] = acc
    pltpu.sync_copy(o_v, out_hbm.at[pl.ds(wid * _L, _L)])


_sc_call = functools.partial(
    pl.kernel,
    out_type=jax.ShapeDtypeStruct((_NW * _L,), jnp.float32),
    mesh=plsc.VectorSubcoreMesh(core_axis_name="c", subcore_axis_name="s"),
    compiler_params=pltpu.CompilerParams(needs_layout_passes=False),
    scratch_types=[
        pltpu.VMEM((_RPW,), jnp.float32),
        pltpu.VMEM((_RPW,), jnp.int32),
        pltpu.VMEM((_CP,), jnp.float32),
        pltpu.VMEM((_L,), jnp.float32),
    ],
)(_sc_body)


def kernel(x, centers, transform_inds):
    xf = x.reshape(_B)
    cf = jnp.concatenate(
        [centers.reshape(_C), jnp.full((_CP - _C,), _INF, jnp.float32)]
    )
    partial = _sc_call(xf, cf, transform_inds)
    return jnp.sum(partial) / _B


# speedup vs baseline: 3.1138x; 2.3330x over previous
import functools

import jax
import jax.numpy as jnp
from jax import lax
from jax.experimental import pallas as pl
from jax.experimental.pallas import tpu as pltpu
from jax.experimental.pallas import tpu_sc as plsc

_B = 16384
_C = 1000
_CP = 1024
_NC = 2
_NS = 16
_NW = _NC * _NS
_RPW = _B // _NW
_L = 16
_G = 4
_MARGIN = 1.0
_INF = float("inf")


def _sc_body(x_hbm, c_hbm, t_hbm, out_hbm, x_v, t_v, c_v, o_v):
    wid = lax.axis_index("s") * _NC + lax.axis_index("c")
    base = wid * _RPW
    pltpu.sync_copy(x_hbm.at[pl.ds(base, _RPW)], x_v)
    pltpu.sync_copy(t_hbm.at[pl.ds(base, _RPW)], t_v)
    pltpu.sync_copy(c_hbm, c_v)

    acc = jnp.zeros((_L,), jnp.float32)
    for g in range(_RPW // (_L * _G)):
        xs = [x_v[pl.ds((g * _G + k) * _L, _L)] for k in range(_G)]
        ts = [t_v[pl.ds((g * _G + k) * _L, _L)] for k in range(_G)]

        def jbody(jj, carry, xs=xs):
            m1s, m2s = list(carry[0]), list(carry[1])
            cblk = c_v[pl.ds(jj * _L, _L)]
            for u in range(_L):
                cj = cblk[u]
                for k in range(_G):
                    d = jnp.abs(xs[k] - cj)
                    m2s[k] = jnp.minimum(m2s[k], jnp.maximum(m1s[k], d))
                    m1s[k] = jnp.minimum(m1s[k], d)
            return tuple(m1s), tuple(m2s)

        init = (
            tuple(jnp.full((_L,), _INF, jnp.float32) for _ in range(_G)),
            tuple(jnp.full((_L,), _INF, jnp.float32) for _ in range(_G)),
        )
        m1s, m2s = lax.fori_loop(0, _CP // _L, jbody, init)
        for k in range(_G):
            own = plsc.load_gather(c_v, [ts[k]])
            d_own = jnp.abs(xs[k] - own)
            push = jnp.where(d_own == m1s[k], m2s[k], m1s[k])
            pull = d_own + _MARGIN
            acc = acc + jnp.maximum(pull - push, 0.0)

    o_v[...] = acc
    pltpu.sync_copy(o_v, out_hbm.at[pl.ds(wid * _L, _L)])


_sc_call = functools.partial(
    pl.kernel,
    out_type=jax.ShapeDtypeStruct((_NW * _L,), jnp.float32),
    mesh=plsc.VectorSubcoreMesh(core_axis_name="c", subcore_axis_name="s"),
    compiler_params=pltpu.CompilerParams(needs_layout_passes=False),
    scratch_types=[
        pltpu.VMEM((_RPW,), jnp.float32),
        pltpu.VMEM((_RPW,), jnp.int32),
        pltpu.VMEM((_CP,), jnp.float32),
        pltpu.VMEM((_L,), jnp.float32),
    ],
)(_sc_body)


def kernel(x, centers, transform_inds):
    xf = x.reshape(_B)
    cf = jnp.concatenate(
        [centers.reshape(_C), jnp.full((_CP - _C,), _INF, jnp.float32)]
    )
    partial = _sc_call(xf, cf, transform_inds)
    return jnp.sum(partial) / _B
